# baseline (speedup 1.0000x reference)
import functools

import jax
import jax.numpy as jnp
from jax import lax
from jax.experimental import pallas as pl
from jax.experimental.pallas import tpu as pltpu
from jax.experimental.pallas import tpu_sc as plsc

_NC = 2
_NS = 16
_NW = _NC * _NS

_B = 16384
_D = 32
_BPW = _B // _NW


def _gather_body(uid_hbm, iid_hbm, ut_hbm, it_hbm, out_u, out_i,
                 uidx_v, iidx_v, urows_v, irows_v, sem_u, sem_i):
    wid = lax.axis_index("s") * _NC + lax.axis_index("c")
    base = wid * _BPW
    pltpu.sync_copy(uid_hbm.at[pl.ds(base, _BPW)], uidx_v)
    pltpu.sync_copy(iid_hbm.at[pl.ds(base, _BPW)], iidx_v)
    cu = pltpu.async_copy(ut_hbm.at[uidx_v], urows_v, sem_u)
    ci = pltpu.async_copy(it_hbm.at[iidx_v], irows_v, sem_i)
    cu.wait()
    ci.wait()
    pltpu.sync_copy(urows_v, out_u.at[pl.ds(base, _BPW)])
    pltpu.sync_copy(irows_v, out_i.at[pl.ds(base, _BPW)])


@functools.cache
def _make_gather():
    return pl.kernel(
        _gather_body,
        out_type=(
            jax.ShapeDtypeStruct((_B, _D), jnp.float32),
            jax.ShapeDtypeStruct((_B, _D), jnp.float32),
        ),
        mesh=plsc.VectorSubcoreMesh(core_axis_name="c", subcore_axis_name="s"),
        scratch_types=[
            pltpu.VMEM((_BPW,), jnp.int32),
            pltpu.VMEM((_BPW,), jnp.int32),
            pltpu.VMEM((_BPW, _D), jnp.float32),
            pltpu.VMEM((_BPW, _D), jnp.float32),
            pltpu.SemaphoreType.DMA,
            pltpu.SemaphoreType.DMA,
        ],
        compiler_params=pltpu.CompilerParams(use_tc_tiling_on_sc=False),
    )


def _mlp_body(u_ref, v_ref, w1u_ref, w1i_ref, b1_ref, w2t_ref, b2_ref,
              w3_ref, b3_ref, out_ref):
    h = u_ref[:] @ w1u_ref[:] + v_ref[:] @ w1i_ref[:] + b1_ref[:]
    h = jnp.maximum(h, 0.0)
    h2 = jnp.maximum(h @ w2t_ref[:] + b2_ref[:], 0.0)
    out_ref[:] = jnp.sum(h2 * w3_ref[:], axis=1) + b3_ref[0]


def _mlp(u, v, w1u_t, w1i_t, b1, w2_t, b2, w3, b3, block_b=2048):
    nb = _B // block_b
    return pl.pallas_call(
        _mlp_body,
        grid=(nb,),
        in_specs=[
            pl.BlockSpec((block_b, _D), lambda i: (i, 0)),
            pl.BlockSpec((block_b, _D), lambda i: (i, 0)),
            pl.BlockSpec(w1u_t.shape, lambda i: (0, 0)),
            pl.BlockSpec(w1i_t.shape, lambda i: (0, 0)),
            pl.BlockSpec(b1.shape, lambda i: (0, 0)),
            pl.BlockSpec(w2_t.shape, lambda i: (0, 0)),
            pl.BlockSpec(b2.shape, lambda i: (0, 0)),
            pl.BlockSpec(w3.shape, lambda i: (0, 0)),
            pl.BlockSpec(b3.shape, lambda i: (0,)),
        ],
        out_specs=pl.BlockSpec((block_b,), lambda i: (i,)),
        out_shape=jax.ShapeDtypeStruct((_B,), jnp.float32),
    )(u, v, w1u_t, w1i_t, b1, w2_t, b2, w3, b3)


def kernel(user_ids, item_ids, user_table, item_table, W1, b1, W2, b2, W3, b3):
    u, v = _make_gather()(user_ids, item_ids, user_table, item_table)
    w1u_t = W1[:, :_D].T
    w1i_t = W1[:, _D:].T
    w2_t = W2.T
    return _mlp(u, v, w1u_t, w1i_t, b1[None, :], w2_t, b2[None, :],
                W3, b3)
